# Initial kernel scaffold; baseline (speedup 1.0000x reference)
import jax
import jax.numpy as jnp
from jax.experimental import pallas as pl


def kernel(x, embed_band, embed_pos):
    raise NotImplementedError("write your pallas kernel here")


# speedup vs baseline: 4.1746x; 4.1746x over previous
import functools

import jax
import jax.numpy as jnp
from jax import lax
from jax.experimental import pallas as pl
from jax.experimental.pallas import tpu as pltpu
from jax.experimental.pallas import tpu_sc as plsc

N_EMBED = 1000
D_MODEL = 128
LENGTH = 200
BATCH = 4096
SCALE = float(D_MODEL) ** 0.5

NUM_CORES = 2
NUM_SUBCORES = 16
NUM_WORKERS = NUM_CORES * NUM_SUBCORES
ROWS_PER_WORKER = BATCH // NUM_WORKERS
LANES = 16
GROUPS = ROWS_PER_WORKER // LANES
L_PAD = 208
G0, G1 = 104, 96


def _embed_body(xt_hbm, band_hbm, pos_hbm, out_hbm, xg, idxg, posv, rowsv, sem):
    c = lax.axis_index("c")
    s = lax.axis_index("s")
    wid = s * NUM_CORES + c
    base = wid * ROWS_PER_WORKER

    pltpu.sync_copy(pos_hbm, posv)

    lane_ids = jnp.arange(LANES, dtype=jnp.int32)

    def group_body(g, carry):
        gbase = base + g * LANES
        pltpu.sync_copy(xt_hbm.at[:, pl.ds(gbase, LANES)], xg)

        def max_body(l, m):
            return jnp.maximum(m, xg[l])

        m = lax.fori_loop(0, LENGTH, max_body, jnp.full((LANES,), -jnp.inf, jnp.float32))

        def quant_body(l, carry2):
            v = xg[l] / m * jnp.float32(N_EMBED - 1)
            v = jnp.where(v < 0.0, 0.0, v)
            plsc.store_scatter(
                idxg, [lane_ids, jnp.full((LANES,), l, jnp.int32)],
                v.astype(jnp.int32),
            )
            return carry2

        lax.fori_loop(0, LENGTH, quant_body, 0)

        def row_body(t, carry3):
            cp0 = pltpu.async_copy(
                band_hbm.at[idxg.at[t, pl.ds(0, G0)]], rowsv.at[pl.ds(0, G0)], sem
            )
            cp1 = pltpu.async_copy(
                band_hbm.at[idxg.at[t, pl.ds(G0, G1)]], rowsv.at[pl.ds(G0, G1)], sem
            )
            cp0.wait()
            cp1.wait()

            def elem_body(l, carry4):
                for j in range(D_MODEL // LANES):
                    sl = pl.ds(j * LANES, LANES)
                    rowsv[l, sl] = rowsv[l, sl] * SCALE + posv[l, sl]
                return carry4

            lax.fori_loop(0, LENGTH, elem_body, 0)

            pltpu.sync_copy(rowsv, out_hbm.at[gbase + t])
            return carry3

        lax.fori_loop(0, LANES, row_body, 0)
        return carry

    lax.fori_loop(0, GROUPS, group_body, 0)


def kernel(x, embed_band, embed_pos):
    xt = x.reshape(BATCH, LENGTH).T
    mesh = plsc.VectorSubcoreMesh(core_axis_name="c", subcore_axis_name="s")
    k = functools.partial(
        pl.kernel,
        mesh=mesh,
        compiler_params=pltpu.CompilerParams(
            use_tc_tiling_on_sc=False, needs_layout_passes=False
        ),
        out_type=jax.ShapeDtypeStruct((BATCH, LENGTH, D_MODEL), jnp.float32),
        scratch_types=[
            pltpu.VMEM((LENGTH, LANES), jnp.float32),
            pltpu.VMEM((LANES, L_PAD), jnp.int32),
            pltpu.VMEM((LENGTH, D_MODEL), jnp.float32),
            pltpu.VMEM((LENGTH, D_MODEL), jnp.float32),
            pltpu.SemaphoreType.DMA,
        ],
    )(_embed_body)
    return k(xt, embed_band, embed_pos)


# speedup vs baseline: 4.3034x; 1.0309x over previous
import functools

import jax
import jax.numpy as jnp
from jax import lax
from jax.experimental import pallas as pl
from jax.experimental.pallas import tpu as pltpu
from jax.experimental.pallas import tpu_sc as plsc

N_EMBED = 1000
D_MODEL = 128
LENGTH = 200
BATCH = 4096
SCALE = float(D_MODEL) ** 0.5

NUM_CORES = 2
NUM_SUBCORES = 16
NUM_WORKERS = NUM_CORES * NUM_SUBCORES
ROWS_PER_WORKER = BATCH // NUM_WORKERS
LANES = 16
GROUPS = ROWS_PER_WORKER // LANES
D_CHUNKS = D_MODEL // LANES


def _embed_body(xt_hbm, band_hbm, pos_hbm, out_hbm, xg, idxt, posv, rowsv, sem):
    c = lax.axis_index("c")
    s = lax.axis_index("s")
    wid = s * NUM_CORES + c
    base = wid * ROWS_PER_WORKER

    pltpu.sync_copy(pos_hbm, posv)

    def group_body(g, carry):
        gbase = base + g * LANES
        pltpu.sync_copy(xt_hbm.at[:, pl.ds(gbase, LANES)], xg)

        def max_body(l, m):
            return jnp.maximum(m, xg[l])

        m = lax.fori_loop(
            0, LENGTH, max_body, jnp.full((LANES,), -jnp.inf, jnp.float32)
        )

        def quant_body(l, carry2):
            v = xg[l] / m * jnp.float32(N_EMBED - 1)
            v = jnp.where(v < 0.0, 0.0, v)
            idxt[l, pl.ds(g * LANES, LANES)] = v.astype(jnp.int32)
            return carry2

        lax.fori_loop(0, LENGTH, quant_body, 0)
        return carry

    lax.fori_loop(0, GROUPS, group_body, 0)

    def pos_body(l, carry):
        cp = pltpu.async_copy(band_hbm.at[idxt.at[l]], rowsv, sem)
        cp.wait()

        pv = [posv[l, pl.ds(j * LANES, LANES)] for j in range(D_CHUNKS)]

        def add_body(t, carry2):
            for j in range(D_CHUNKS):
                sl = pl.ds(j * LANES, LANES)
                rowsv[t, sl] = rowsv[t, sl] * SCALE + pv[j]
            return carry2

        lax.fori_loop(0, ROWS_PER_WORKER, add_body, 0)

        pltpu.sync_copy(rowsv, out_hbm.at[pl.ds(base, ROWS_PER_WORKER), l])
        return carry

    lax.fori_loop(0, LENGTH, pos_body, 0)


def kernel(x, embed_band, embed_pos):
    xt = x.reshape(BATCH, LENGTH).T
    mesh = plsc.VectorSubcoreMesh(core_axis_name="c", subcore_axis_name="s")
    k = functools.partial(
        pl.kernel,
        mesh=mesh,
        compiler_params=pltpu.CompilerParams(
            use_tc_tiling_on_sc=False, needs_layout_passes=False
        ),
        out_type=jax.ShapeDtypeStruct((BATCH, LENGTH, D_MODEL), jnp.float32),
        scratch_types=[
            pltpu.VMEM((LENGTH, LANES), jnp.float32),
            pltpu.VMEM((LENGTH, ROWS_PER_WORKER), jnp.int32),
            pltpu.VMEM((LENGTH, D_MODEL), jnp.float32),
            pltpu.VMEM((ROWS_PER_WORKER, D_MODEL), jnp.float32),
            pltpu.SemaphoreType.DMA,
        ],
    )(_embed_body)
    return k(xt, embed_band, embed_pos)


# speedup vs baseline: 5.1589x; 1.1988x over previous
import functools

import jax
import jax.numpy as jnp
from jax import lax
from jax.experimental import pallas as pl
from jax.experimental.pallas import tpu as pltpu
from jax.experimental.pallas import tpu_sc as plsc

N_EMBED = 1000
D_MODEL = 128
LENGTH = 200
BATCH = 4096
SCALE = float(D_MODEL) ** 0.5

NUM_CORES = 2
NUM_SUBCORES = 16
NUM_WORKERS = NUM_CORES * NUM_SUBCORES
ROWS_PER_WORKER = BATCH // NUM_WORKERS
LANES = 16
GROUPS = ROWS_PER_WORKER // LANES
D_CHUNKS = D_MODEL // LANES


def _embed_body(
    xt_hbm, band_hbm, pos_hbm, out_hbm,
    xg, idxt, posv, buf0, buf1, sg0, sg1, so0, so1,
):
    c = lax.axis_index("c")
    s = lax.axis_index("s")
    wid = s * NUM_CORES + c
    base = wid * ROWS_PER_WORKER

    pltpu.sync_copy(pos_hbm, posv)

    def group_body(g, carry):
        gbase = base + g * LANES
        pltpu.sync_copy(xt_hbm.at[:, pl.ds(gbase, LANES)], xg)

        def max_body(l, m):
            return jnp.maximum(m, xg[l])

        m = lax.fori_loop(
            0, LENGTH, max_body, jnp.full((LANES,), -jnp.inf, jnp.float32)
        )

        def quant_body(l, carry2):
            v = xg[l] / m * jnp.float32(N_EMBED - 1)
            v = jnp.where(v < 0.0, 0.0, v)
            idxt[l, pl.ds(g * LANES, LANES)] = v.astype(jnp.int32)
            return carry2

        lax.fori_loop(0, LENGTH, quant_body, 0)
        return carry

    lax.fori_loop(0, GROUPS, group_body, 0)

    out_slice = out_hbm.at[pl.ds(base, ROWS_PER_WORKER)]

    def compute(buf, l):
        pv = [posv[l, pl.ds(j * LANES, LANES)] for j in range(D_CHUNKS)]

        def add_body(t, carry2):
            for j in range(D_CHUNKS):
                sl = pl.ds(j * LANES, LANES)
                buf[t, sl] = buf[t, sl] * SCALE + pv[j]
            return carry2

        lax.fori_loop(0, ROWS_PER_WORKER, add_body, 0)

    def gather_issue(l, buf, sem):
        pltpu.async_copy(band_hbm.at[idxt.at[l]], buf, sem)

    def gather_wait(l, buf, sem):
        pltpu.make_async_copy(band_hbm.at[idxt.at[l]], buf, sem).wait()

    def out_issue(l, buf, sem):
        pltpu.async_copy(buf, out_slice.at[:, l], sem)

    def out_wait(l, buf, sem):
        pltpu.make_async_copy(buf, out_slice.at[:, l], sem).wait()

    gather_issue(0, buf0, sg0)

    def pair_body(k, carry):
        l0 = 2 * k
        l1 = l0 + 1
        gather_wait(l0, buf0, sg0)

        @pl.when(k >= 1)
        def _():
            out_wait(l1, buf1, so1)

        gather_issue(l1, buf1, sg1)
        compute(buf0, l0)
        out_issue(l0, buf0, so0)
        gather_wait(l1, buf1, sg1)
        compute(buf1, l1)
        out_issue(l1, buf1, so1)
        out_wait(l0, buf0, so0)

        @pl.when(k < LENGTH // 2 - 1)
        def _():
            gather_issue(l0 + 2, buf0, sg0)

        return carry

    lax.fori_loop(0, LENGTH // 2, pair_body, 0)
    out_wait(LENGTH - 1, buf1, so1)


def kernel(x, embed_band, embed_pos):
    xt = x.reshape(BATCH, LENGTH).T
    mesh = plsc.VectorSubcoreMesh(core_axis_name="c", subcore_axis_name="s")
    k = functools.partial(
        pl.kernel,
        mesh=mesh,
        compiler_params=pltpu.CompilerParams(
            use_tc_tiling_on_sc=False, needs_layout_passes=False
        ),
        out_type=jax.ShapeDtypeStruct((BATCH, LENGTH, D_MODEL), jnp.float32),
        scratch_types=[
            pltpu.VMEM((LENGTH, LANES), jnp.float32),
            pltpu.VMEM((LENGTH, ROWS_PER_WORKER), jnp.int32),
            pltpu.VMEM((LENGTH, D_MODEL), jnp.float32),
            pltpu.VMEM((ROWS_PER_WORKER, D_MODEL), jnp.float32),
            pltpu.VMEM((ROWS_PER_WORKER, D_MODEL), jnp.float32),
            pltpu.SemaphoreType.DMA,
            pltpu.SemaphoreType.DMA,
            pltpu.SemaphoreType.DMA,
            pltpu.SemaphoreType.DMA,
        ],
    )(_embed_body)
    return k(xt, embed_band, embed_pos)


# speedup vs baseline: 10.1987x; 1.9769x over previous
import functools

import jax
import jax.numpy as jnp
from jax import lax
from jax.experimental import pallas as pl
from jax.experimental.pallas import tpu as pltpu
from jax.experimental.pallas import tpu_sc as plsc

N_EMBED = 1000
D_MODEL = 128
LENGTH = 200
BATCH = 4096
SCALE = float(D_MODEL) ** 0.5

NUM_CORES = 2
NUM_SUBCORES = 16
NUM_WORKERS = NUM_CORES * NUM_SUBCORES
ROWS_PER_WORKER = BATCH // NUM_WORKERS
LANES = 16
GROUPS = ROWS_PER_WORKER // LANES
D_CHUNKS = D_MODEL // LANES


ROWS_PER_TILE = 63


def _embed_body(
    xt_hbm, band_hbm, pos_hbm, out_hbm,
    xg, idxt, posv, buf0, buf1, table_sh, sg0, sg1, so0, so1,
):
    c = lax.axis_index("c")
    s = lax.axis_index("s")
    wid = s * NUM_CORES + c
    base = wid * ROWS_PER_WORKER

    tstart = jnp.minimum(s * ROWS_PER_TILE, N_EMBED - ROWS_PER_TILE)
    pltpu.sync_copy(
        band_hbm.at[pl.ds(tstart, ROWS_PER_TILE)],
        table_sh.at[pl.ds(tstart, ROWS_PER_TILE)],
    )

    pltpu.sync_copy(pos_hbm, posv)

    def group_body(g, carry):
        gbase = base + g * LANES
        pltpu.sync_copy(xt_hbm.at[:, pl.ds(gbase, LANES)], xg)

        def max_body(l, m):
            return jnp.maximum(m, xg[l])

        m = lax.fori_loop(
            0, LENGTH, max_body, jnp.full((LANES,), -jnp.inf, jnp.float32)
        )

        def quant_body(l, carry2):
            v = xg[l] / m * jnp.float32(N_EMBED - 1)
            v = jnp.where(v < 0.0, 0.0, v)
            idxt[l, pl.ds(g * LANES, LANES)] = v.astype(jnp.int32)
            return carry2

        lax.fori_loop(0, LENGTH, quant_body, 0)
        return carry

    lax.fori_loop(0, GROUPS, group_body, 0)

    plsc.subcore_barrier()

    out_slice = out_hbm.at[pl.ds(base, ROWS_PER_WORKER)]

    def compute(buf, l):
        pv = [posv[l, pl.ds(j * LANES, LANES)] for j in range(D_CHUNKS)]

        def add_body(t, carry2):
            for j in range(D_CHUNKS):
                sl = pl.ds(j * LANES, LANES)
                buf[t, sl] = buf[t, sl] * SCALE + pv[j]
            return carry2

        lax.fori_loop(0, ROWS_PER_WORKER, add_body, 0)

    def gather_issue(l, buf, sem):
        pltpu.async_copy(table_sh.at[idxt.at[l]], buf, sem)

    def gather_wait(l, buf, sem):
        pltpu.make_async_copy(table_sh.at[idxt.at[l]], buf, sem).wait()

    def out_issue(l, buf, sem):
        pltpu.async_copy(buf, out_slice.at[:, l], sem)

    def out_wait(l, buf, sem):
        pltpu.make_async_copy(buf, out_slice.at[:, l], sem).wait()

    gather_issue(0, buf0, sg0)

    def pair_body(k, carry):
        l0 = 2 * k
        l1 = l0 + 1
        gather_wait(l0, buf0, sg0)

        @pl.when(k >= 1)
        def _():
            out_wait(l1, buf1, so1)

        gather_issue(l1, buf1, sg1)
        compute(buf0, l0)
        out_issue(l0, buf0, so0)
        gather_wait(l1, buf1, sg1)
        compute(buf1, l1)
        out_issue(l1, buf1, so1)
        out_wait(l0, buf0, so0)

        @pl.when(k < LENGTH // 2 - 1)
        def _():
            gather_issue(l0 + 2, buf0, sg0)

        return carry

    lax.fori_loop(0, LENGTH // 2, pair_body, 0)
    out_wait(LENGTH - 1, buf1, so1)


def kernel(x, embed_band, embed_pos):
    xt = x.reshape(BATCH, LENGTH).T
    mesh = plsc.VectorSubcoreMesh(core_axis_name="c", subcore_axis_name="s")
    k = functools.partial(
        pl.kernel,
        mesh=mesh,
        compiler_params=pltpu.CompilerParams(
            use_tc_tiling_on_sc=False, needs_layout_passes=False
        ),
        out_type=jax.ShapeDtypeStruct((BATCH, LENGTH, D_MODEL), jnp.float32),
        scratch_types=[
            pltpu.VMEM((LENGTH, LANES), jnp.float32),
            pltpu.VMEM((LENGTH, ROWS_PER_WORKER), jnp.int32),
            pltpu.VMEM((LENGTH, D_MODEL), jnp.float32),
            pltpu.VMEM((ROWS_PER_WORKER, D_MODEL), jnp.float32),
            pltpu.VMEM((ROWS_PER_WORKER, D_MODEL), jnp.float32),
            pltpu.VMEM_SHARED((N_EMBED, D_MODEL), jnp.float32),
            pltpu.SemaphoreType.DMA,
            pltpu.SemaphoreType.DMA,
            pltpu.SemaphoreType.DMA,
            pltpu.SemaphoreType.DMA,
        ],
    )(_embed_body)
    return k(xt, embed_band, embed_pos)


# speedup vs baseline: 10.2248x; 1.0026x over previous
import functools

import jax
import jax.numpy as jnp
from jax import lax
from jax.experimental import pallas as pl
from jax.experimental.pallas import tpu as pltpu
from jax.experimental.pallas import tpu_sc as plsc

N_EMBED = 1000
D_MODEL = 128
LENGTH = 200
BATCH = 4096
SCALE = float(D_MODEL) ** 0.5

NUM_CORES = 2
NUM_SUBCORES = 16
NUM_WORKERS = NUM_CORES * NUM_SUBCORES
ROWS_PER_WORKER = BATCH // NUM_WORKERS
LANES = 16
GROUPS = ROWS_PER_WORKER // LANES
D_CHUNKS = D_MODEL // LANES


ROWS_PER_TILE = 63


def _embed_body(
    xt_hbm, band_hbm, pos_hbm, out_hbm,
    xg, idxt, posv, buf0, buf1, table_sh, sg0, sg1, so0, so1,
):
    c = lax.axis_index("c")
    s = lax.axis_index("s")
    wid = s * NUM_CORES + c
    base = wid * ROWS_PER_WORKER

    @pl.when(s == 0)
    def _():
        pltpu.sync_copy(band_hbm, table_sh)

    pltpu.sync_copy(pos_hbm, posv)

    def group_body(g, carry):
        gbase = base + g * LANES
        pltpu.sync_copy(xt_hbm.at[:, pl.ds(gbase, LANES)], xg)

        def max_body(l, m):
            return jnp.maximum(m, xg[l])

        m = lax.fori_loop(
            0, LENGTH, max_body, jnp.full((LANES,), -jnp.inf, jnp.float32)
        )

        def quant_body(l, carry2):
            v = xg[l] / m * jnp.float32(N_EMBED - 1)
            v = jnp.where(v < 0.0, 0.0, v)
            idxt[l, pl.ds(g * LANES, LANES)] = v.astype(jnp.int32)
            return carry2

        lax.fori_loop(0, LENGTH, quant_body, 0)
        return carry

    lax.fori_loop(0, GROUPS, group_body, 0)

    plsc.subcore_barrier()

    out_slice = out_hbm.at[pl.ds(base, ROWS_PER_WORKER)]

    def compute(buf, l):
        pv = [posv[l, pl.ds(j * LANES, LANES)] for j in range(D_CHUNKS)]

        def add_body(t, carry2):
            for j in range(D_CHUNKS):
                sl = pl.ds(j * LANES, LANES)
                buf[t, sl] = buf[t, sl] * SCALE + pv[j]
            return carry2

        lax.fori_loop(0, ROWS_PER_WORKER, add_body, 0)

    def gather_issue(l, buf, sem):
        pltpu.async_copy(table_sh.at[idxt.at[l]], buf, sem)

    def gather_wait(l, buf, sem):
        pltpu.make_async_copy(table_sh.at[idxt.at[l]], buf, sem).wait()

    def out_issue(l, buf, sem):
        pltpu.async_copy(buf, out_slice.at[:, l], sem)

    def out_wait(l, buf, sem):
        pltpu.make_async_copy(buf, out_slice.at[:, l], sem).wait()

    gather_issue(0, buf0, sg0)

    def pair_body(k, carry):
        l0 = 2 * k
        l1 = l0 + 1
        gather_wait(l0, buf0, sg0)

        @pl.when(k >= 1)
        def _():
            out_wait(l1, buf1, so1)

        gather_issue(l1, buf1, sg1)
        compute(buf0, l0)
        out_issue(l0, buf0, so0)
        gather_wait(l1, buf1, sg1)
        compute(buf1, l1)
        out_issue(l1, buf1, so1)
        out_wait(l0, buf0, so0)

        @pl.when(k < LENGTH // 2 - 1)
        def _():
            gather_issue(l0 + 2, buf0, sg0)

        return carry

    lax.fori_loop(0, LENGTH // 2, pair_body, 0)
    out_wait(LENGTH - 1, buf1, so1)


def kernel(x, embed_band, embed_pos):
    xt = x.reshape(BATCH, LENGTH).T
    mesh = plsc.VectorSubcoreMesh(core_axis_name="c", subcore_axis_name="s")
    k = functools.partial(
        pl.kernel,
        mesh=mesh,
        compiler_params=pltpu.CompilerParams(
            use_tc_tiling_on_sc=False, needs_layout_passes=False
        ),
        out_type=jax.ShapeDtypeStruct((BATCH, LENGTH, D_MODEL), jnp.float32),
        scratch_types=[
            pltpu.VMEM((LENGTH, LANES), jnp.float32),
            pltpu.VMEM((LENGTH, ROWS_PER_WORKER), jnp.int32),
            pltpu.VMEM((LENGTH, D_MODEL), jnp.float32),
            pltpu.VMEM((ROWS_PER_WORKER, D_MODEL), jnp.float32),
            pltpu.VMEM((ROWS_PER_WORKER, D_MODEL), jnp.float32),
            pltpu.VMEM_SHARED((N_EMBED, D_MODEL), jnp.float32),
            pltpu.SemaphoreType.DMA,
            pltpu.SemaphoreType.DMA,
            pltpu.SemaphoreType.DMA,
            pltpu.SemaphoreType.DMA,
        ],
    )(_embed_body)
    return k(xt, embed_band, embed_pos)


# speedup vs baseline: 10.4269x; 1.0198x over previous
import functools

import jax
import jax.numpy as jnp
from jax import lax
from jax.experimental import pallas as pl
from jax.experimental.pallas import tpu as pltpu
from jax.experimental.pallas import tpu_sc as plsc

N_EMBED = 1000
D_MODEL = 128
LENGTH = 200
BATCH = 4096
SCALE = float(D_MODEL) ** 0.5

NUM_CORES = 2
NUM_SUBCORES = 16
NUM_WORKERS = NUM_CORES * NUM_SUBCORES
ROWS_PER_WORKER = BATCH // NUM_WORKERS
LANES = 16
GROUPS = ROWS_PER_WORKER // LANES
D_CHUNKS = D_MODEL // LANES


ROWS_PER_TILE = 63


def _embed_body(
    xt_hbm, band_hbm, pos_hbm, out_hbm,
    xg, idxt, posv, buf0, buf1, table_sh, sg0, sg1, so0, so1,
):
    c = lax.axis_index("c")
    s = lax.axis_index("s")
    wid = s * NUM_CORES + c
    base = wid * ROWS_PER_WORKER

    @pl.when(s == 0)
    def _():
        pltpu.async_copy(band_hbm, table_sh, sg0)

    pltpu.sync_copy(pos_hbm, posv)

    def group_body(g, carry):
        gbase = base + g * LANES
        pltpu.sync_copy(xt_hbm.at[:, pl.ds(gbase, LANES)], xg)

        def max_body(l, m):
            return jnp.maximum(m, xg[l])

        m = lax.fori_loop(
            0, LENGTH, max_body, jnp.full((LANES,), -jnp.inf, jnp.float32)
        )

        def quant_body(l, carry2):
            v = xg[l] / m * jnp.float32(N_EMBED - 1)
            v = jnp.where(v < 0.0, 0.0, v)
            idxt[l, pl.ds(g * LANES, LANES)] = v.astype(jnp.int32)
            return carry2

        lax.fori_loop(0, LENGTH, quant_body, 0)
        return carry

    lax.fori_loop(0, GROUPS, group_body, 0)

    @pl.when(s == 0)
    def _():
        pltpu.make_async_copy(band_hbm, table_sh, sg0).wait()

    plsc.subcore_barrier()

    out_slice = out_hbm.at[pl.ds(base, ROWS_PER_WORKER)]

    def compute(buf, l):
        pv = [posv[l, pl.ds(j * LANES, LANES)] for j in range(D_CHUNKS)]

        @plsc.parallel_loop(0, ROWS_PER_WORKER, unroll=4)
        def add_body(t):
            for j in range(D_CHUNKS):
                sl = pl.ds(j * LANES, LANES)
                buf[t, sl] = buf[t, sl] * SCALE + pv[j]

    def gather_issue(l, buf, sem):
        pltpu.async_copy(table_sh.at[idxt.at[l]], buf, sem)

    def gather_wait(l, buf, sem):
        pltpu.make_async_copy(table_sh.at[idxt.at[l]], buf, sem).wait()

    def out_issue(l, buf, sem):
        pltpu.async_copy(buf, out_slice.at[:, l], sem)

    def out_wait(l, buf, sem):
        pltpu.make_async_copy(buf, out_slice.at[:, l], sem).wait()

    gather_issue(0, buf0, sg0)

    def pair_body(k, carry):
        l0 = 2 * k
        l1 = l0 + 1
        gather_wait(l0, buf0, sg0)

        @pl.when(k >= 1)
        def _():
            out_wait(l1, buf1, so1)

        gather_issue(l1, buf1, sg1)
        compute(buf0, l0)
        out_issue(l0, buf0, so0)
        gather_wait(l1, buf1, sg1)
        compute(buf1, l1)
        out_issue(l1, buf1, so1)
        out_wait(l0, buf0, so0)

        @pl.when(k < LENGTH // 2 - 1)
        def _():
            gather_issue(l0 + 2, buf0, sg0)

        return carry

    lax.fori_loop(0, LENGTH // 2, pair_body, 0)
    out_wait(LENGTH - 1, buf1, so1)


def kernel(x, embed_band, embed_pos):
    xt = x.reshape(BATCH, LENGTH).T
    mesh = plsc.VectorSubcoreMesh(core_axis_name="c", subcore_axis_name="s")
    k = functools.partial(
        pl.kernel,
        mesh=mesh,
        compiler_params=pltpu.CompilerParams(
            use_tc_tiling_on_sc=False, needs_layout_passes=False
        ),
        out_type=jax.ShapeDtypeStruct((BATCH, LENGTH, D_MODEL), jnp.float32),
        scratch_types=[
            pltpu.VMEM((LENGTH, LANES), jnp.float32),
            pltpu.VMEM((LENGTH, ROWS_PER_WORKER), jnp.int32),
            pltpu.VMEM((LENGTH, D_MODEL), jnp.float32),
            pltpu.VMEM((ROWS_PER_WORKER, D_MODEL), jnp.float32),
            pltpu.VMEM((ROWS_PER_WORKER, D_MODEL), jnp.float32),
            pltpu.VMEM_SHARED((N_EMBED, D_MODEL), jnp.float32),
            pltpu.SemaphoreType.DMA,
            pltpu.SemaphoreType.DMA,
            pltpu.SemaphoreType.DMA,
            pltpu.SemaphoreType.DMA,
        ],
    )(_embed_body)
    return k(xt, embed_band, embed_pos)
